# baseline (speedup 1.0000x reference)
import functools

import jax
import jax.numpy as jnp
from jax import lax
from jax.experimental import pallas as pl
from jax.experimental.pallas import tpu as pltpu
from jax.experimental.pallas import tpu_sc as plsc

NUM_TX = 4
NUM_RE = 12
NUM_SYM = 14
NUM_PRB = 132
NUM_RX = 8
BATCH = 16
GRID = NUM_RE * NUM_SYM
TASK_ROWS = NUM_PRB * GRID
A_ROWS_PER_B = 792 * NUM_TX
PRB_PER_CHUNK = 12
NUM_CHUNKS = NUM_PRB // PRB_PER_CHUNK
CHUNK_ROWS = GRID * PRB_PER_CHUNK
ROWS_PER_DMA = 112
DMAS_PER_CHUNK = CHUNK_ROWS // ROWS_PER_DMA
IDX_VECS = TASK_ROWS // 16


def _prep_body(ofp_ref, scp_ref, pe_ref, idx_ref):
    t = pl.program_id(0)
    iy = lax.broadcasted_iota(jnp.int32, (NUM_RE, NUM_SYM), 0)
    ix = lax.broadcasted_iota(jnp.int32, (NUM_RE, NUM_SYM), 1)
    q = ix * NUM_RE + iy
    rr = q // NUM_SYM
    ss = q % NUM_SYM
    big = jnp.full((NUM_RE, NUM_SYM), 127, jnp.int32)
    best = big
    nn = jnp.zeros((NUM_RE, NUM_SYM), jnp.int32)
    msc = big
    mof = big
    for k in range(12):
        c, a = k // 2, k % 2
        scv = scp_ref[t, c]
        ofv = ofp_ref[t, a]
        dsc = jnp.abs(rr - scv)
        dof = jnp.abs(ss - ofv)
        d = dsc + dof
        nn = jnp.where(d < best, k, nn)
        best = jnp.minimum(best, d)
        msc = jnp.minimum(msc, dsc)
        mof = jnp.minimum(mof, dof)

    def _norm(v):
        vc = v - jnp.sum(v) / 168.0
        sd = jnp.sqrt(jnp.sum(vc * vc) / 167.0)
        return jnp.where(sd > 0.0, vc / sd, vc)

    nof = _norm(mof.astype(jnp.float32))
    nsc = _norm(msc.astype(jnp.float32))
    pe_ref[0] = jnp.concatenate([nof[:, :, None], nsc[:, :, None]], axis=-1)

    rowrel = ((nn // 6) * 396 + (nn % 6) // 2) * 4 + t
    p3 = lax.broadcasted_iota(jnp.int32, (NUM_PRB, NUM_RE, NUM_SYM), 0)
    idx_ref[0] = rowrel[None] + 12 * p3


def _prep(dmrs_ofdm_pos, dmrs_subcarrier_pos):
    return pl.pallas_call(
        _prep_body,
        grid=(NUM_TX,),
        in_specs=[
            pl.BlockSpec(memory_space=pltpu.SMEM),
            pl.BlockSpec(memory_space=pltpu.SMEM),
        ],
        out_specs=[
            pl.BlockSpec((1, NUM_RE, NUM_SYM, 2), lambda t: (t, 0, 0, 0)),
            pl.BlockSpec((1, NUM_PRB, NUM_RE, NUM_SYM),
                         lambda t: (t, 0, 0, 0)),
        ],
        out_shape=[
            jax.ShapeDtypeStruct((NUM_TX, NUM_RE, NUM_SYM, 2), jnp.float32),
            jax.ShapeDtypeStruct((NUM_TX, NUM_PRB, NUM_RE, NUM_SYM),
                                 jnp.int32),
        ],
    )(dmrs_ofdm_pos, dmrs_subcarrier_pos)


def _avg_body(h_ref, a_ref):
    x = h_ref[0]
    a_ref[0] = 0.5 * (x[:, 0, :] + x[:, 1, :])


def _avg(h2):
    return pl.pallas_call(
        _avg_body,
        grid=(BATCH,),
        in_specs=[pl.BlockSpec((1, 792, 2, NUM_TX * NUM_RX),
                               lambda b: (b, 0, 0, 0))],
        out_specs=pl.BlockSpec((1, 792, NUM_TX * NUM_RX),
                               lambda b: (b, 0, 0)),
        out_shape=jax.ShapeDtypeStruct((BATCH, 792, NUM_TX * NUM_RX),
                                       jnp.float32),
    )(h2)


@functools.cache
def _sc_gather_fn():
    mesh = plsc.VectorSubcoreMesh(core_axis_name="c", subcore_axis_name="s")
    return pl.kernel(
        _sc_gather,
        out_type=jax.ShapeDtypeStruct((BATCH * NUM_TX, TASK_ROWS, NUM_RX),
                                      jnp.float32),
        mesh=mesh,
        compiler_params=pltpu.CompilerParams(use_tc_tiling_on_sc=False),
        scratch_types=[
            pltpu.VMEM((TASK_ROWS,), jnp.int32),
            pltpu.VMEM((2, CHUNK_ROWS, NUM_RX), jnp.float32),
            pltpu.SemaphoreType.DMA,
            pltpu.SemaphoreType.DMA,
            pltpu.SemaphoreType.DMA,
        ],
    )


def _sc_gather(a_hbm, idx_hbm, out_hbm, idx_v, obuf, gsem, wsem0, wsem1):
    wid = lax.axis_index("s") * 2 + lax.axis_index("c")
    wsems = (wsem0, wsem1)
    for rep in range(2):
        task = wid * 2 + rep
        b = task // NUM_TX
        t = task % NUM_TX
        pltpu.sync_copy(idx_hbm.at[t], idx_v)
        basev = jnp.full((16,), b * A_ROWS_PER_B, jnp.int32)

        def _addbase(w, _, basev=basev):
            idx_v[pl.ds(w * 16, 16)] = idx_v[pl.ds(w * 16, 16)] + basev
            return 0

        lax.fori_loop(0, IDX_VECS, _addbase, 0)

        pending = [None, None]
        for ch in range(NUM_CHUNKS):
            slot = ch % 2
            if pending[slot] is not None:
                pending[slot].wait()
                pending[slot] = None

            def _gbody(g, _, slot=slot, ch=ch):
                off = ch * CHUNK_ROWS + g * ROWS_PER_DMA
                pltpu.async_copy(
                    a_hbm.at[idx_v.at[pl.ds(off, ROWS_PER_DMA)]],
                    obuf.at[slot, pl.ds(g * ROWS_PER_DMA, ROWS_PER_DMA)],
                    gsem)
                return 0

            lax.fori_loop(0, DMAS_PER_CHUNK, _gbody, 0)
            pltpu.make_async_copy(
                out_hbm.at[task, pl.ds(0, CHUNK_ROWS)], obuf.at[slot], gsem
            ).wait()
            pending[slot] = pltpu.async_copy(
                obuf.at[slot],
                out_hbm.at[task, pl.ds(ch * CHUNK_ROWS, CHUNK_ROWS)],
                wsems[slot])
        for slot in range(2):
            if pending[slot] is not None:
                pending[slot].wait()


def kernel(y, h_hat_ls, dmrs_ofdm_pos, dmrs_subcarrier_pos):
    del y
    pe_small, idx4 = _prep(dmrs_ofdm_pos, dmrs_subcarrier_pos)
    a = _avg(h_hat_ls.reshape(BATCH, 792, 2, NUM_TX * NUM_RX))
    out = _sc_gather_fn()(
        a.reshape(BATCH * 792 * NUM_TX, NUM_RX),
        idx4.reshape(NUM_TX, TASK_ROWS),
    )
    h_hat = out.reshape(BATCH, NUM_TX, NUM_PRB * NUM_RE, NUM_SYM, NUM_RX)
    pe = jnp.broadcast_to(
        pe_small[:, None], (NUM_TX, NUM_PRB, NUM_RE, NUM_SYM, 2)
    ).reshape(NUM_TX, NUM_PRB * NUM_RE, NUM_SYM, 2)
    return (h_hat, pe)


# speedup vs baseline: 1.7739x; 1.7739x over previous
import functools

import jax
import jax.numpy as jnp
from jax import lax
from jax.experimental import pallas as pl
from jax.experimental.pallas import tpu as pltpu
from jax.experimental.pallas import tpu_sc as plsc

NUM_TX = 4
NUM_RE = 12
NUM_SYM = 14
NUM_PRB = 132
NUM_RX = 8
BATCH = 16
GRID = NUM_RE * NUM_SYM
TASK_ROWS = NUM_PRB * GRID
A_ROWS_PER_B = 792 * NUM_TX
PRB_PER_CHUNK = 12
NUM_CHUNKS = NUM_PRB // PRB_PER_CHUNK
CHUNK_ROWS = GRID * PRB_PER_CHUNK
ROWS_PER_DMA = 112
DMAS_PER_CHUNK = CHUNK_ROWS // ROWS_PER_DMA
IDX_VECS = TASK_ROWS // 16


def _prep_body(ofp_ref, scp_ref, pe_ref, idx_ref):
    t = pl.program_id(0)
    iy = lax.broadcasted_iota(jnp.int32, (NUM_RE, NUM_SYM), 0)
    ix = lax.broadcasted_iota(jnp.int32, (NUM_RE, NUM_SYM), 1)
    q = ix * NUM_RE + iy
    rr = q // NUM_SYM
    ss = q % NUM_SYM
    big = jnp.full((NUM_RE, NUM_SYM), 127, jnp.int32)
    best = big
    nn = jnp.zeros((NUM_RE, NUM_SYM), jnp.int32)
    msc = big
    mof = big
    for k in range(12):
        c, a = k // 2, k % 2
        scv = scp_ref[t, c]
        ofv = ofp_ref[t, a]
        dsc = jnp.abs(rr - scv)
        dof = jnp.abs(ss - ofv)
        d = dsc + dof
        nn = jnp.where(d < best, k, nn)
        best = jnp.minimum(best, d)
        msc = jnp.minimum(msc, dsc)
        mof = jnp.minimum(mof, dof)

    def _norm(v):
        vc = v - jnp.sum(v) / 168.0
        sd = jnp.sqrt(jnp.sum(vc * vc) / 167.0)
        return jnp.where(sd > 0.0, vc / sd, vc)

    nof = _norm(mof.astype(jnp.float32))
    nsc = _norm(msc.astype(jnp.float32))
    pe_ref[0] = jnp.concatenate([nof[:, :, None], nsc[:, :, None]], axis=-1)

    ixT = lax.broadcasted_iota(jnp.int32, (NUM_SYM, NUM_RE), 0)
    iyT = lax.broadcasted_iota(jnp.int32, (NUM_SYM, NUM_RE), 1)
    qT = ixT * NUM_RE + iyT
    rrT = qT // NUM_SYM
    ssT = qT % NUM_SYM
    bigT = jnp.full((NUM_SYM, NUM_RE), 127, jnp.int32)
    bestT = bigT
    nnT = jnp.zeros((NUM_SYM, NUM_RE), jnp.int32)
    for k in range(12):
        c, a = k // 2, k % 2
        dT = jnp.abs(rrT - scp_ref[t, c]) + jnp.abs(ssT - ofp_ref[t, a])
        nnT = jnp.where(dT < bestT, k, nnT)
        bestT = jnp.minimum(bestT, dT)

    rowrelT = ((nnT // 6) * 396 + (nnT % 6) // 2) * 4 + t
    p3 = lax.broadcasted_iota(jnp.int32, (NUM_SYM, NUM_PRB, NUM_RE), 1)
    idx_ref[0] = rowrelT[:, None, :] + 12 * p3


def _prep(dmrs_ofdm_pos, dmrs_subcarrier_pos):
    return pl.pallas_call(
        _prep_body,
        grid=(NUM_TX,),
        in_specs=[
            pl.BlockSpec(memory_space=pltpu.SMEM),
            pl.BlockSpec(memory_space=pltpu.SMEM),
        ],
        out_specs=[
            pl.BlockSpec((1, NUM_RE, NUM_SYM, 2), lambda t: (t, 0, 0, 0)),
            pl.BlockSpec((1, NUM_SYM, NUM_PRB, NUM_RE),
                         lambda t: (t, 0, 0, 0)),
        ],
        out_shape=[
            jax.ShapeDtypeStruct((NUM_TX, NUM_RE, NUM_SYM, 2), jnp.float32),
            jax.ShapeDtypeStruct((NUM_TX, NUM_SYM, NUM_PRB, NUM_RE),
                                 jnp.int32),
        ],
    )(dmrs_ofdm_pos, dmrs_subcarrier_pos)


def _avg_body(h_ref, a_ref):
    x = h_ref[0]
    a_ref[0] = 0.5 * (x[:, 0, :] + x[:, 1, :])


def _avg(h2):
    return pl.pallas_call(
        _avg_body,
        grid=(BATCH,),
        in_specs=[pl.BlockSpec((1, 792, 2, NUM_TX * NUM_RX),
                               lambda b: (b, 0, 0, 0))],
        out_specs=pl.BlockSpec((1, 792, NUM_TX * NUM_RX),
                               lambda b: (b, 0, 0)),
        out_shape=jax.ShapeDtypeStruct((BATCH, 792, NUM_TX * NUM_RX),
                                       jnp.float32),
    )(h2)


@functools.cache
def _sc_gather_fn():
    mesh = plsc.VectorSubcoreMesh(core_axis_name="c", subcore_axis_name="s")
    return pl.kernel(
        _sc_gather,
        out_type=jax.ShapeDtypeStruct((BATCH * NUM_TX, TASK_ROWS, NUM_RX),
                                      jnp.float32),
        mesh=mesh,
        compiler_params=pltpu.CompilerParams(use_tc_tiling_on_sc=False),
        scratch_types=[
            pltpu.VMEM((TASK_ROWS,), jnp.int32),
            pltpu.VMEM((2, CHUNK_ROWS, NUM_RX), jnp.float32),
            pltpu.SemaphoreType.DMA,
            pltpu.SemaphoreType.DMA,
            pltpu.SemaphoreType.DMA,
        ],
    )


def _sc_gather(a_hbm, idx_hbm, out_hbm, idx_v, obuf, gsem, wsem0, wsem1):
    wid = lax.axis_index("s") * 2 + lax.axis_index("c")
    wsems = (wsem0, wsem1)
    for rep in range(2):
        task = wid * 2 + rep
        b = task // NUM_TX
        t = task % NUM_TX
        pltpu.sync_copy(idx_hbm.at[t], idx_v)
        basev = jnp.full((16,), b * A_ROWS_PER_B, jnp.int32)

        def _addbase(w, _, basev=basev):
            idx_v[pl.ds(w * 16, 16)] = idx_v[pl.ds(w * 16, 16)] + basev
            return 0

        lax.fori_loop(0, IDX_VECS, _addbase, 0)

        pending = [None, None]
        for ch in range(NUM_CHUNKS):
            slot = ch % 2
            if pending[slot] is not None:
                pending[slot].wait()
                pending[slot] = None

            def _gbody(g, _, slot=slot, ch=ch):
                off = ch * CHUNK_ROWS + g * ROWS_PER_DMA
                pltpu.async_copy(
                    a_hbm.at[idx_v.at[pl.ds(off, ROWS_PER_DMA)]],
                    obuf.at[slot, pl.ds(g * ROWS_PER_DMA, ROWS_PER_DMA)],
                    gsem)
                return 0

            lax.fori_loop(0, DMAS_PER_CHUNK, _gbody, 0)
            pltpu.make_async_copy(
                out_hbm.at[task, pl.ds(0, CHUNK_ROWS)], obuf.at[slot], gsem
            ).wait()
            pending[slot] = pltpu.async_copy(
                obuf.at[slot],
                out_hbm.at[task, pl.ds(ch * CHUNK_ROWS, CHUNK_ROWS)],
                wsems[slot])
        for slot in range(2):
            if pending[slot] is not None:
                pending[slot].wait()


def kernel(y, h_hat_ls, dmrs_ofdm_pos, dmrs_subcarrier_pos):
    del y
    pe_small, idx4 = _prep(dmrs_ofdm_pos, dmrs_subcarrier_pos)
    a = _avg(h_hat_ls.reshape(BATCH, 792, 2, NUM_TX * NUM_RX))
    out = _sc_gather_fn()(
        a.reshape(BATCH * 792 * NUM_TX, NUM_RX),
        idx4.reshape(NUM_TX, TASK_ROWS),
    )
    h5 = out.reshape(BATCH, NUM_TX, NUM_SYM, NUM_PRB * NUM_RE, NUM_RX)
    h_hat = jnp.transpose(h5, (0, 1, 3, 2, 4))
    pe = jnp.broadcast_to(
        pe_small[:, None], (NUM_TX, NUM_PRB, NUM_RE, NUM_SYM, 2)
    ).reshape(NUM_TX, NUM_PRB * NUM_RE, NUM_SYM, 2)
    return (h_hat, pe)
